# baseline (speedup 1.0000x reference)
import functools

import jax
import jax.numpy as jnp
from jax import lax
from jax.experimental import pallas as pl
from jax.experimental.pallas import tpu as pltpu
from jax.experimental.pallas import tpu_sc as plsc

_NUM_LEVELS = 3
_DECAY = 0.8

_TR = 256
_CB = 2048


def _level_starts(n):
    sizes = []
    remaining = n
    for i in range(_NUM_LEVELS):
        if i == _NUM_LEVELS - 1:
            sizes.append(remaining)
        else:
            ls = int(remaining * (1.0 - _DECAY) * (_DECAY ** i))
            sizes.append(ls)
            remaining -= ls
    c1 = sizes[0]
    c2 = sizes[0] + sizes[1]
    return c1, c2


def _dst_body(c1, c2, imp_ref, dst_ref):
    imp = imp_ref[...]
    r_dim, l_dim = imp.shape
    key = lax.bitcast_convert_type(imp, jnp.int32)
    lane = lax.broadcasted_iota(jnp.int32, (r_dim, l_dim), 1)
    row = lax.broadcasted_iota(jnp.int32, (r_dim, l_dim), 0)
    i_idx = (row * l_dim + lane).astype(jnp.float32)
    m_le = (row <= lane).astype(jnp.float32)
    p_lt = (lane < row).astype(jnp.float32)

    def incl_prefix(e):
        within = jnp.dot(e, m_le, preferred_element_type=jnp.float32)
        s = jnp.sum(e, axis=1, keepdims=True)
        rowoff = jnp.dot(p_lt, s, preferred_element_type=jnp.float32)
        return within + rowoff

    def topc(c):
        def body(t, carry):
            cand, above, rem = carry
            b = 29 - t
            bit = jnp.right_shift(key, b) & 1
            ones_m = cand * bit.astype(jnp.float32)
            n1 = jnp.sum(ones_m)
            take = n1 >= rem
            cand2 = jnp.where(take, ones_m, cand - ones_m)
            above2 = jnp.where(take, above, above + ones_m)
            rem2 = jnp.where(take, rem, rem - n1)
            return cand2, above2, rem2

        init = (jnp.ones((r_dim, l_dim), jnp.float32),
                jnp.zeros((r_dim, l_dim), jnp.float32),
                jnp.float32(c))
        cand, above, rem = lax.fori_loop(0, 30, body, init)
        incl = incl_prefix(cand)
        return above + cand * (incl <= rem).astype(jnp.float32)

    top1 = topc(c1)
    top2 = topc(c2)
    l0 = top1
    l1 = top2 - top1
    l2 = 1.0 - top2
    e0 = incl_prefix(l0) - l0
    e1 = incl_prefix(l1) - l1
    dst = (l0 * e0 + l1 * (c1 + e1)
           + l2 * (c2 + i_idx - e0 - e1))
    dst_ref[...] = dst.astype(jnp.int32)


def _compute_dst(imp_flat, c1, c2):
    n = imp_flat.shape[0]
    a = imp_flat.reshape(n // 128, 128)
    out = pl.pallas_call(
        functools.partial(_dst_body, c1, c2),
        out_shape=jax.ShapeDtypeStruct((n // 128, 128), jnp.int32),
    )(a)
    return out.reshape(n)


_NC, _NS = 2, 16
_NW = _NC * _NS
_CH = 64


def _sc_mesh():
    return plsc.VectorSubcoreMesh(core_axis_name="c", subcore_axis_name="s")


def _sc_scatter_rows(fk, fv, rank, rank_n):
    n, h = fk.shape
    rpw = n // _NW

    @functools.partial(
        pl.kernel,
        out_type=jax.ShapeDtypeStruct((2 * n, h), jnp.float32),
        mesh=_sc_mesh(),
        scratch_types=[
            pltpu.VMEM((_CH,), jnp.int32),
            pltpu.VMEM((_CH, h), jnp.float32),
            pltpu.SemaphoreType.DMA,
        ],
    )
    def scatter_k(fk_hbm, fv_hbm, rk_hbm, rkn_hbm, x_hbm, idx_v, rows_v, sem):
        wid = lax.axis_index("s") * _NC + lax.axis_index("c")
        base = wid * rpw
        for src_hbm, i_hbm in ((fk_hbm, rk_hbm), (fv_hbm, rkn_hbm)):
            for c in range(rpw // _CH):
                b = base + c * _CH
                pltpu.sync_copy(i_hbm.at[pl.ds(b, _CH)], idx_v)
                pltpu.sync_copy(src_hbm.at[pl.ds(b, _CH)], rows_v)
                pltpu.async_copy(rows_v, x_hbm.at[idx_v], sem).wait()

    return scatter_k(fk, fv, rank, rank_n)


def _sc_gather_rows(y, rank, rank_n):
    n2, h = y.shape
    n = n2 // 2
    rpw = n // _NW

    @functools.partial(
        pl.kernel,
        out_type=(jax.ShapeDtypeStruct((n, h), jnp.float32),
                  jax.ShapeDtypeStruct((n, h), jnp.float32)),
        mesh=_sc_mesh(),
        scratch_types=[
            pltpu.VMEM((_CH,), jnp.int32),
            pltpu.VMEM((_CH, h), jnp.float32),
            pltpu.SemaphoreType.DMA,
        ],
    )
    def gather_k(y_hbm, rk_hbm, rkn_hbm, ck_hbm, cv_hbm, idx_v, rows_v, sem):
        wid = lax.axis_index("s") * _NC + lax.axis_index("c")
        base = wid * rpw
        for dst_hbm, i_hbm in ((ck_hbm, rk_hbm), (cv_hbm, rkn_hbm)):
            for c in range(rpw // _CH):
                b = base + c * _CH
                pltpu.sync_copy(i_hbm.at[pl.ds(b, _CH)], idx_v)
                pltpu.async_copy(y_hbm.at[idx_v], rows_v, sem).wait()
                pltpu.sync_copy(rows_v, dst_hbm.at[pl.ds(b, _CH)])

    return gather_k(y, rank, rank_n)


def _mlp_body(c1, c2, x_ref, we0_ref, be0_ref, wd0_ref, bd0_ref,
              we1_ref, be1_ref, wd1_ref, bd1_ref, o_ref):
    i = pl.program_id(0)
    im = lax.rem(i, jnp.int32(64))
    t1 = c1 // _TR
    t2 = c2 // _TR
    x = x_ref[...]

    def bdot(v, w_ref):
        return jnp.dot(v.astype(jnp.bfloat16), w_ref[...].astype(jnp.bfloat16),
                       preferred_element_type=jnp.float32)

    def enc0(v):
        return jnp.maximum(bdot(v, we0_ref) + be0_ref[...], 0.0)

    def dec0(v):
        return jnp.maximum(bdot(v, wd0_ref) + bd0_ref[...], 0.0)

    def mid(v):
        h = jnp.maximum(bdot(v, we1_ref) + be1_ref[...], 0.0)
        return jnp.maximum(bdot(h, wd1_ref) + bd1_ref[...], 0.0)

    rows = lax.broadcasted_iota(jnp.int32, (_TR, 1), 0) + im * _TR

    @pl.when(im < t1)
    def _():
        o_ref[...] = x

    @pl.when(im == t1)
    def _():
        h = enc0(x)
        o_ref[...] = jnp.where(rows < c1, x, dec0(h))

    @pl.when((im > t1) & (im < t2))
    def _():
        o_ref[...] = dec0(enc0(x))

    @pl.when(im == t2)
    def _():
        h = enc0(x)
        o_ref[...] = jnp.where(rows < c2, dec0(h), dec0(mid(h)))

    @pl.when(im > t2)
    def _():
        o_ref[...] = dec0(mid(enc0(x)))


def _run_mlp(x, c1, c2, we0, be0, wd0, bd0, we1p, be1p, wd1p, bd1p):
    n2, h = x.shape
    d1 = we0.shape[1]
    d2p = we1p.shape[1]
    body = functools.partial(_mlp_body, c1, c2)
    return pl.pallas_call(
        body,
        grid=(n2 // _TR,),
        in_specs=[
            pl.BlockSpec((_TR, h), lambda i: (i, 0)),
            pl.BlockSpec((h, d1), lambda i: (0, 0)),
            pl.BlockSpec((1, d1), lambda i: (0, 0)),
            pl.BlockSpec((d1, h), lambda i: (0, 0)),
            pl.BlockSpec((1, h), lambda i: (0, 0)),
            pl.BlockSpec((d1, d2p), lambda i: (0, 0)),
            pl.BlockSpec((1, d2p), lambda i: (0, 0)),
            pl.BlockSpec((d2p, d1), lambda i: (0, 0)),
            pl.BlockSpec((1, d1), lambda i: (0, 0)),
        ],
        out_specs=pl.BlockSpec((_TR, h), lambda i: (i, 0)),
        out_shape=jax.ShapeDtypeStruct((n2, h), jnp.float32),
    )(x, we0, be0, wd0, bd0, we1p, be1p, wd1p, bd1p)


def kernel(keys, values, importance, We0, be0, We1, be1, We2, be2,
           Wd0, bd0, Wd1, bd1, Wd2, bd2):
    bsz, s, h = keys.shape
    n = bsz * s
    c1, c2 = _level_starts(n)

    fk = keys.reshape(n, h)
    fv = values.reshape(n, h)
    imp = importance.reshape(n)

    rank = _compute_dst(imp, c1, c2)
    rank_n = rank + n

    d2 = We1.shape[1]
    d2p = 256
    we1p = jnp.pad(We1, ((0, 0), (0, d2p - d2)))
    be1p = jnp.pad(be1, ((0, d2p - d2),)).reshape(1, d2p)
    wd1p = jnp.pad(Wd1, ((0, d2p - d2), (0, 0)))

    x = _sc_scatter_rows(fk, fv, rank, rank_n)

    y = _run_mlp(x, c1, c2, We0, be0.reshape(1, -1), Wd0, bd0.reshape(1, -1),
                 we1p, be1p, wd1p, bd1.reshape(1, -1))

    ck, cv = _sc_gather_rows(y, rank, rank_n)
    return ck.reshape(bsz, s, h), cv.reshape(bsz, s, h)


# speedup vs baseline: 1.2673x; 1.2673x over previous
import functools

import jax
import jax.numpy as jnp
from jax import lax
from jax.experimental import pallas as pl
from jax.experimental.pallas import tpu as pltpu
from jax.experimental.pallas import tpu_sc as plsc

_NUM_LEVELS = 3
_DECAY = 0.8

_TR = 256
_CB = 2048


def _level_starts(n):
    sizes = []
    remaining = n
    for i in range(_NUM_LEVELS):
        if i == _NUM_LEVELS - 1:
            sizes.append(remaining)
        else:
            ls = int(remaining * (1.0 - _DECAY) * (_DECAY ** i))
            sizes.append(ls)
            remaining -= ls
    c1 = sizes[0]
    c2 = sizes[0] + sizes[1]
    return c1, c2


def _dst_body(c1, c2, imp_ref, dst_ref):
    imp = imp_ref[...]
    r_dim, l_dim = imp.shape
    key = lax.bitcast_convert_type(imp, jnp.int32)
    lane = lax.broadcasted_iota(jnp.int32, (r_dim, l_dim), 1)
    row = lax.broadcasted_iota(jnp.int32, (r_dim, l_dim), 0)
    i_idx = (row * l_dim + lane).astype(jnp.float32)
    m_le = (row <= lane).astype(jnp.float32)
    p_lt = (lane < row).astype(jnp.float32)

    def incl_prefix(e):
        within = jnp.dot(e, m_le, preferred_element_type=jnp.float32)
        s = jnp.sum(e, axis=1, keepdims=True)
        rowoff = jnp.dot(p_lt, s, preferred_element_type=jnp.float32)
        return within + rowoff

    def topc(c):
        def body(t, carry):
            cand, above, rem = carry
            b = 29 - t
            bit = jnp.right_shift(key, b) & 1
            ones_m = cand * bit.astype(jnp.float32)
            n1 = jnp.sum(ones_m)
            take = n1 >= rem
            cand2 = jnp.where(take, ones_m, cand - ones_m)
            above2 = jnp.where(take, above, above + ones_m)
            rem2 = jnp.where(take, rem, rem - n1)
            return cand2, above2, rem2

        init = (jnp.ones((r_dim, l_dim), jnp.float32),
                jnp.zeros((r_dim, l_dim), jnp.float32),
                jnp.float32(c))
        cand, above, rem = lax.fori_loop(0, 30, body, init)
        incl = incl_prefix(cand)
        return above + cand * (incl <= rem).astype(jnp.float32)

    top1 = topc(c1)
    top2 = topc(c2)
    l0 = top1
    l1 = top2 - top1
    l2 = 1.0 - top2
    e0 = incl_prefix(l0) - l0
    e1 = incl_prefix(l1) - l1
    dst = (l0 * e0 + l1 * (c1 + e1)
           + l2 * (c2 + i_idx - e0 - e1))
    dst_ref[...] = dst.astype(jnp.int32)


def _compute_dst(imp_flat, c1, c2):
    n = imp_flat.shape[0]
    a = imp_flat.reshape(n // 128, 128)
    out = pl.pallas_call(
        functools.partial(_dst_body, c1, c2),
        out_shape=jax.ShapeDtypeStruct((n // 128, 128), jnp.int32),
    )(a)
    return out.reshape(n)


_NC, _NS = 2, 16
_NW = _NC * _NS
_CH = 64


def _sc_mesh():
    return plsc.VectorSubcoreMesh(core_axis_name="c", subcore_axis_name="s")


def _sc_scatter_rows(src, dst_idx):
    n, h = src.shape
    rpw = n // _NW

    @functools.partial(
        pl.kernel,
        out_type=jax.ShapeDtypeStruct((n, h), jnp.float32),
        mesh=_sc_mesh(),
        scratch_types=[
            pltpu.VMEM((_CH,), jnp.int32),
            pltpu.VMEM((_CH, h), jnp.float32),
            pltpu.SemaphoreType.DMA,
        ],
    )
    def scatter_k(src_hbm, i_hbm, x_hbm, idx_v, rows_v, sem):
        wid = lax.axis_index("s") * _NC + lax.axis_index("c")
        base = wid * rpw
        for c in range(rpw // _CH):
            b = base + c * _CH
            pltpu.sync_copy(i_hbm.at[pl.ds(b, _CH)], idx_v)
            pltpu.sync_copy(src_hbm.at[pl.ds(b, _CH)], rows_v)
            pltpu.async_copy(rows_v, x_hbm.at[idx_v], sem).wait()

    return scatter_k(src, dst_idx)


def _sc_gather_rows(y, src_idx):
    n, h = y.shape
    rpw = n // _NW

    @functools.partial(
        pl.kernel,
        out_type=jax.ShapeDtypeStruct((n, h), jnp.float32),
        mesh=_sc_mesh(),
        scratch_types=[
            pltpu.VMEM((_CH,), jnp.int32),
            pltpu.VMEM((_CH, h), jnp.float32),
            pltpu.SemaphoreType.DMA,
        ],
    )
    def gather_k(y_hbm, i_hbm, out_hbm, idx_v, rows_v, sem):
        wid = lax.axis_index("s") * _NC + lax.axis_index("c")
        base = wid * rpw
        for c in range(rpw // _CH):
            b = base + c * _CH
            pltpu.sync_copy(i_hbm.at[pl.ds(b, _CH)], idx_v)
            pltpu.async_copy(y_hbm.at[idx_v], rows_v, sem).wait()
            pltpu.sync_copy(rows_v, out_hbm.at[pl.ds(b, _CH)])

    return gather_k(y, src_idx)


def _mlp_body(c1, c2, x_ref, we0_ref, be0_ref, wd0_ref, bd0_ref,
              we1_ref, be1_ref, wd1_ref, bd1_ref, o_ref):
    t1 = c1 // _TR
    t2 = c2 // _TR
    im = pl.program_id(0) + t1
    x = x_ref[...]

    def bdot(v, w_ref):
        return jnp.dot(v.astype(jnp.bfloat16), w_ref[...].astype(jnp.bfloat16),
                       preferred_element_type=jnp.float32)

    def enc0(v):
        return jnp.maximum(bdot(v, we0_ref) + be0_ref[...], 0.0)

    def dec0(v):
        return jnp.maximum(bdot(v, wd0_ref) + bd0_ref[...], 0.0)

    def mid(v):
        h = jnp.maximum(bdot(v, we1_ref) + be1_ref[...], 0.0)
        return jnp.maximum(bdot(h, wd1_ref) + bd1_ref[...], 0.0)

    rows = lax.broadcasted_iota(jnp.int32, (_TR, 1), 0) + im * _TR

    @pl.when(im == t1)
    def _():
        h = enc0(x)
        o_ref[...] = jnp.where(rows < c1, x, dec0(h))

    @pl.when((im > t1) & (im < t2))
    def _():
        o_ref[...] = dec0(enc0(x))

    @pl.when(im == t2)
    def _():
        h = enc0(x)
        o_ref[...] = jnp.where(rows < c2, dec0(h), dec0(mid(h)))

    @pl.when(im > t2)
    def _():
        o_ref[...] = dec0(mid(enc0(x)))


def _run_mlp(x, c1, c2, we0, be0, wd0, bd0, we1p, be1p, wd1p, bd1p):
    n, h = x.shape
    d1 = we0.shape[1]
    d2p = we1p.shape[1]
    t1 = c1 // _TR
    body = functools.partial(_mlp_body, c1, c2)
    return pl.pallas_call(
        body,
        grid=(n // _TR - t1,),
        in_specs=[
            pl.BlockSpec((_TR, h), lambda i: (i + t1, 0)),
            pl.BlockSpec((h, d1), lambda i: (0, 0)),
            pl.BlockSpec((1, d1), lambda i: (0, 0)),
            pl.BlockSpec((d1, h), lambda i: (0, 0)),
            pl.BlockSpec((1, h), lambda i: (0, 0)),
            pl.BlockSpec((d1, d2p), lambda i: (0, 0)),
            pl.BlockSpec((1, d2p), lambda i: (0, 0)),
            pl.BlockSpec((d2p, d1), lambda i: (0, 0)),
            pl.BlockSpec((1, d1), lambda i: (0, 0)),
        ],
        out_specs=pl.BlockSpec((_TR, h), lambda i: (i + t1, 0)),
        out_shape=jax.ShapeDtypeStruct((n, h), jnp.float32),
        input_output_aliases={0: 0},
    )(x, we0, be0, wd0, bd0, we1p, be1p, wd1p, bd1p)


def kernel(keys, values, importance, We0, be0, We1, be1, We2, be2,
           Wd0, bd0, Wd1, bd1, Wd2, bd2):
    bsz, s, h = keys.shape
    n = bsz * s
    c1, c2 = _level_starts(n)

    fk = keys.reshape(n, h)
    fv = values.reshape(n, h)
    imp = importance.reshape(n)

    dst = _compute_dst(imp, c1, c2)

    d2 = We1.shape[1]
    d2p = 256
    we1p = jnp.pad(We1, ((0, 0), (0, d2p - d2)))
    be1p = jnp.pad(be1, ((0, d2p - d2),)).reshape(1, d2p)
    wd1p = jnp.pad(Wd1, ((0, d2p - d2), (0, 0)))

    def run(wargs, xk):
        return _run_mlp(xk, c1, c2, *wargs)

    wargs = (We0, be0.reshape(1, -1), Wd0, bd0.reshape(1, -1),
             we1p, be1p, wd1p, bd1.reshape(1, -1))

    xk = _sc_scatter_rows(fk, dst)
    xv = _sc_scatter_rows(fv, dst)
    yk = run(wargs, xk)
    yv = run(wargs, xv)
    ck = _sc_gather_rows(yk, dst)
    cv = _sc_gather_rows(yv, dst)
    return ck.reshape(bsz, s, h), cv.reshape(bsz, s, h)


# speedup vs baseline: 1.2700x; 1.0022x over previous
import functools

import jax
import jax.numpy as jnp
from jax import lax
from jax.experimental import pallas as pl
from jax.experimental.pallas import tpu as pltpu
from jax.experimental.pallas import tpu_sc as plsc

_NUM_LEVELS = 3
_DECAY = 0.8

_TR = 256
_CB = 2048


def _level_starts(n):
    sizes = []
    remaining = n
    for i in range(_NUM_LEVELS):
        if i == _NUM_LEVELS - 1:
            sizes.append(remaining)
        else:
            ls = int(remaining * (1.0 - _DECAY) * (_DECAY ** i))
            sizes.append(ls)
            remaining -= ls
    c1 = sizes[0]
    c2 = sizes[0] + sizes[1]
    return c1, c2


def _dst_body(c1, c2, imp_ref, dst_ref):
    imp = imp_ref[...]
    r_dim, l_dim = imp.shape
    key = lax.bitcast_convert_type(imp, jnp.int32)
    lane = lax.broadcasted_iota(jnp.int32, (r_dim, l_dim), 1)
    row = lax.broadcasted_iota(jnp.int32, (r_dim, l_dim), 0)
    i_idx = (row * l_dim + lane).astype(jnp.float32)
    m_le = (row <= lane).astype(jnp.float32)
    p_lt = (lane < row).astype(jnp.float32)

    def incl_prefix(e):
        within = jnp.dot(e, m_le, preferred_element_type=jnp.float32)
        s = jnp.sum(e, axis=1, keepdims=True)
        rowoff = jnp.dot(p_lt, s, preferred_element_type=jnp.float32)
        return within + rowoff

    def topc(c):
        def body(t, carry):
            cand, above, rem = carry
            b = 29 - t
            bit = jnp.right_shift(key, b) & 1
            ones_m = cand * bit.astype(jnp.float32)
            n1 = jnp.sum(ones_m)
            take = n1 >= rem
            cand2 = jnp.where(take, ones_m, cand - ones_m)
            above2 = jnp.where(take, above, above + ones_m)
            rem2 = jnp.where(take, rem, rem - n1)
            return cand2, above2, rem2

        init = (jnp.ones((r_dim, l_dim), jnp.float32),
                jnp.zeros((r_dim, l_dim), jnp.float32),
                jnp.float32(c))
        cand, above, rem = lax.fori_loop(0, 30, body, init)
        incl = incl_prefix(cand)
        return above + cand * (incl <= rem).astype(jnp.float32)

    top1 = topc(c1)
    top2 = topc(c2)
    l0 = top1
    l1 = top2 - top1
    l2 = 1.0 - top2
    e0 = incl_prefix(l0) - l0
    e1 = incl_prefix(l1) - l1
    dst = (l0 * e0 + l1 * (c1 + e1)
           + l2 * (c2 + i_idx - e0 - e1))
    dst_ref[...] = dst.astype(jnp.int32)


def _compute_dst(imp_flat, c1, c2):
    n = imp_flat.shape[0]
    a = imp_flat.reshape(n // 128, 128)
    out = pl.pallas_call(
        functools.partial(_dst_body, c1, c2),
        out_shape=jax.ShapeDtypeStruct((n // 128, 128), jnp.int32),
    )(a)
    return out.reshape(n)


_NC, _NS = 2, 16
_NW = _NC * _NS
_CH = 64


def _sc_mesh():
    return plsc.VectorSubcoreMesh(core_axis_name="c", subcore_axis_name="s")


def _sc_scatter_rows(src, dst_idx):
    n, h = src.shape
    rpw = n // _NW

    @functools.partial(
        pl.kernel,
        out_type=jax.ShapeDtypeStruct((n, h), jnp.float32),
        mesh=_sc_mesh(),
        scratch_types=[
            pltpu.VMEM((_CH,), jnp.int32),
            pltpu.VMEM((_CH, h), jnp.float32),
            pltpu.SemaphoreType.DMA,
        ],
    )
    def scatter_k(src_hbm, i_hbm, x_hbm, idx_v, rows_v, sem):
        wid = lax.axis_index("s") * _NC + lax.axis_index("c")
        base = wid * rpw
        for c in range(rpw // _CH):
            b = base + c * _CH
            pltpu.sync_copy(i_hbm.at[pl.ds(b, _CH)], idx_v)
            pltpu.sync_copy(src_hbm.at[pl.ds(b, _CH)], rows_v)
            pltpu.async_copy(rows_v, x_hbm.at[idx_v], sem).wait()

    return scatter_k(src, dst_idx)


def _sc_gather_rows(y, src_idx):
    n, h = y.shape
    rpw = n // _NW

    @functools.partial(
        pl.kernel,
        out_type=jax.ShapeDtypeStruct((n, h), jnp.float32),
        mesh=_sc_mesh(),
        scratch_types=[
            pltpu.VMEM((_CH,), jnp.int32),
            pltpu.VMEM((_CH, h), jnp.float32),
            pltpu.SemaphoreType.DMA,
        ],
    )
    def gather_k(y_hbm, i_hbm, out_hbm, idx_v, rows_v, sem):
        wid = lax.axis_index("s") * _NC + lax.axis_index("c")
        base = wid * rpw
        for c in range(rpw // _CH):
            b = base + c * _CH
            pltpu.sync_copy(i_hbm.at[pl.ds(b, _CH)], idx_v)
            pltpu.async_copy(y_hbm.at[idx_v], rows_v, sem).wait()
            pltpu.sync_copy(rows_v, out_hbm.at[pl.ds(b, _CH)])

    return gather_k(y, src_idx)


def _mlp_body(c1, c2, x_ref, we0_ref, be0_ref, wd0_ref, bd0_ref,
              we1_ref, be1_ref, wd1_ref, bd1_ref, o_ref):
    t1 = c1 // _TR
    t2 = c2 // _TR
    im = pl.program_id(0) + t1
    x = x_ref[...]

    def bdot(v, w_ref):
        return jnp.dot(v.astype(jnp.bfloat16), w_ref[...],
                       preferred_element_type=jnp.float32)

    def enc0(v):
        return jnp.maximum(bdot(v, we0_ref) + be0_ref[...], 0.0)

    def dec0(v):
        return jnp.maximum(bdot(v, wd0_ref) + bd0_ref[...], 0.0)

    def mid(v):
        h = jnp.maximum(bdot(v, we1_ref) + be1_ref[...], 0.0)
        return jnp.maximum(bdot(h, wd1_ref) + bd1_ref[...], 0.0)

    rows = lax.broadcasted_iota(jnp.int32, (_TR, 1), 0) + im * _TR

    @pl.when(im == t1)
    def _():
        h = enc0(x)
        o_ref[...] = jnp.where(rows < c1, x, dec0(h))

    @pl.when((im > t1) & (im < t2))
    def _():
        o_ref[...] = dec0(enc0(x))

    @pl.when(im == t2)
    def _():
        h = enc0(x)
        o_ref[...] = jnp.where(rows < c2, dec0(h), dec0(mid(h)))

    @pl.when(im > t2)
    def _():
        o_ref[...] = dec0(mid(enc0(x)))


def _run_mlp(x, c1, c2, we0, be0, wd0, bd0, we1p, be1p, wd1p, bd1p):
    n, h = x.shape
    d1 = we0.shape[1]
    d2p = we1p.shape[1]
    t1 = c1 // _TR
    body = functools.partial(_mlp_body, c1, c2)
    return pl.pallas_call(
        body,
        grid=(n // _TR - t1,),
        in_specs=[
            pl.BlockSpec((_TR, h), lambda i: (i + t1, 0)),
            pl.BlockSpec((h, d1), lambda i: (0, 0)),
            pl.BlockSpec((1, d1), lambda i: (0, 0)),
            pl.BlockSpec((d1, h), lambda i: (0, 0)),
            pl.BlockSpec((1, h), lambda i: (0, 0)),
            pl.BlockSpec((d1, d2p), lambda i: (0, 0)),
            pl.BlockSpec((1, d2p), lambda i: (0, 0)),
            pl.BlockSpec((d2p, d1), lambda i: (0, 0)),
            pl.BlockSpec((1, d1), lambda i: (0, 0)),
        ],
        out_specs=pl.BlockSpec((_TR, h), lambda i: (i + t1, 0)),
        out_shape=jax.ShapeDtypeStruct((n, h), jnp.float32),
        input_output_aliases={0: 0},
    )(x, we0, be0, wd0, bd0, we1p, be1p, wd1p, bd1p)


def kernel(keys, values, importance, We0, be0, We1, be1, We2, be2,
           Wd0, bd0, Wd1, bd1, Wd2, bd2):
    bsz, s, h = keys.shape
    n = bsz * s
    c1, c2 = _level_starts(n)

    fk = keys.reshape(n, h)
    fv = values.reshape(n, h)
    imp = importance.reshape(n)

    dst = _compute_dst(imp, c1, c2)

    d2 = We1.shape[1]
    d2p = 256
    we1p = jnp.pad(We1, ((0, 0), (0, d2p - d2)))
    be1p = jnp.pad(be1, ((0, d2p - d2),)).reshape(1, d2p)
    wd1p = jnp.pad(Wd1, ((0, d2p - d2), (0, 0)))

    def run(wargs, xk):
        return _run_mlp(xk, c1, c2, *wargs)

    wargs = (We0.astype(jnp.bfloat16), be0.reshape(1, -1),
             Wd0.astype(jnp.bfloat16), bd0.reshape(1, -1),
             we1p.astype(jnp.bfloat16), be1p,
             wd1p.astype(jnp.bfloat16), bd1.reshape(1, -1))

    xk = _sc_scatter_rows(fk, dst)
    xv = _sc_scatter_rows(fv, dst)
    yk = run(wargs, xk)
    yv = run(wargs, xv)
    ck = _sc_gather_rows(yk, dst)
    cv = _sc_gather_rows(yv, dst)
    return ck.reshape(bsz, s, h), cv.reshape(bsz, s, h)
